# baseline (speedup 1.0000x reference)
import functools

import jax
import jax.numpy as jnp
from jax import lax
from jax.experimental import pallas as pl
from jax.experimental.pallas import tpu as pltpu
from jax.experimental.pallas import tpu_sc as plsc

VOCAB = 100000
D = 32
CTX = 20
BATCH = 1024

NC = 2
NS = 16
NW = NC * NS
NIDX = CTX * BATCH
ICH = 4096
NCHUNK = NIDX // ICH
GFIRE = 5
GITER = NCHUNK // GFIRE

_mesh = plsc.VectorSubcoreMesh(core_axis_name="c", subcore_axis_name="s")


@functools.partial(
    pl.kernel,
    mesh=_mesh,
    out_type=jax.ShapeDtypeStruct((D, BATCH), jnp.float32),
    scratch_types=[
        pltpu.VMEM((NCHUNK, ICH), jnp.int32),
        pltpu.VMEM((NIDX,), jnp.float32),
        pltpu.VMEM((BATCH,), jnp.float32),
        pltpu.SemaphoreType.DMA,
    ],
    compiler_params=pltpu.CompilerParams(use_tc_tiling_on_sc=False),
)
def _gather_sum(idx_hbm, tablet_hbm, out_hbm, idx_v, vals_v, acc_v, sem):
    wid = lax.axis_index("s") * NC + lax.axis_index("c")
    pltpu.sync_copy(idx_hbm, idx_v)
    row = tablet_hbm.at[wid]

    copies = [
        pltpu.async_copy(
            row.at[idx_v.at[j]],
            vals_v.at[pl.ds(j * ICH, ICH)],
            sem,
        )
        for j in range(NCHUNK)
    ]
    CPC = ICH // BATCH
    for j in range(NCHUNK):
        copies[j].wait()

        def rbody(g, carry, j=j):
            o = g * 16
            acc = vals_v[pl.ds(j * ICH + o, 16)]
            for c in range(1, CPC):
                acc = acc + vals_v[pl.ds(j * ICH + c * BATCH + o, 16)]
            if j > 0:
                acc = acc + acc_v[pl.ds(o, 16)]
            acc_v[pl.ds(o, 16)] = acc
            return carry

        lax.fori_loop(0, BATCH // 16, rbody, 0)
    pltpu.sync_copy(acc_v, out_hbm.at[wid])


VT = 2048
GRID = (VOCAB + VT - 1) // VT
NSPLIT = 4
CH = VT // NSPLIT
TAIL = VOCAB - (GRID - 1) * VT
TFULL = TAIL // CH
TREM = TAIL - TFULL * CH


def _proj_body(wt_ref, es_ref, b_ref, out_hbm, obuf, sem_o):
    i = pl.program_id(0)
    n = pl.num_programs(0)
    slot = lax.rem(i, 2)

    def chunk_copy(step, k, rows):
        s = lax.rem(step, 2)
        base = pl.multiple_of(step * VT, VT) + k * CH
        return pltpu.make_async_copy(
            obuf.at[s, pl.ds(k * CH, rows)],
            out_hbm.at[pl.ds(base, rows)],
            sem_o.at[s, k],
        )

    def start_out(step, last):
        nfull = TFULL if last else NSPLIT
        for k in range(nfull):
            chunk_copy(step, k, CH).start()
        if last:
            chunk_copy(step, TFULL, TREM).start()

    def drain_out(step, last):
        nfull = TFULL if last else NSPLIT
        for k in range(nfull):
            chunk_copy(step, k, CH).wait()
        if last:
            chunk_copy(step, TFULL, TREM).wait()

    @pl.when(i >= 2)
    def _():
        drain_out(i - 2, False)

    obuf[slot] = (
        lax.dot_general(
            wt_ref[...],
            es_ref[...],
            (((0,), (0,)), ((), ())),
            preferred_element_type=jnp.float32,
        )
        + b_ref[...][:, None]
    )

    @pl.when(i < n - 1)
    def _():
        start_out(i, False)

    @pl.when(i == n - 1)
    def _():
        start_out(i, True)
        drain_out(i - 1, False)
        drain_out(i, True)


def kernel(context_words, emb_table, W, b):
    idx = jnp.asarray(context_words, jnp.int32).reshape(NCHUNK, ICH)
    es_t = _gather_sum(idx, emb_table.T)

    out_t = pl.pallas_call(
        _proj_body,
        grid=(GRID,),
        in_specs=[
            pl.BlockSpec((D, VT), lambda i: (0, i)),
            pl.BlockSpec((D, BATCH), lambda i: (0, 0)),
            pl.BlockSpec((VT,), lambda i: (i,)),
        ],
        out_specs=pl.BlockSpec(memory_space=pl.ANY),
        out_shape=jax.ShapeDtypeStruct((VOCAB, BATCH), jnp.float32),
        scratch_shapes=[
            pltpu.VMEM((2, VT, BATCH), jnp.float32),
            pltpu.SemaphoreType.DMA((2, NSPLIT)),
        ],
    )(W.T, es_t, b)
    return out_t.T


# speedup vs baseline: 1.0058x; 1.0058x over previous
import functools

import jax
import jax.numpy as jnp
from jax import lax
from jax.experimental import pallas as pl
from jax.experimental.pallas import tpu as pltpu
from jax.experimental.pallas import tpu_sc as plsc

VOCAB = 100000
D = 32
CTX = 20
BATCH = 1024

NC = 2
NS = 16
NW = NC * NS
NIDX = CTX * BATCH
ICH = 4096
NCHUNK = NIDX // ICH

_mesh = plsc.VectorSubcoreMesh(core_axis_name="c", subcore_axis_name="s")


@functools.partial(
    pl.kernel,
    mesh=_mesh,
    out_type=jax.ShapeDtypeStruct((D, BATCH), jnp.float32),
    scratch_types=[
        pltpu.VMEM((NCHUNK, ICH), jnp.int32),
        pltpu.VMEM((NIDX,), jnp.float32),
        pltpu.VMEM((BATCH,), jnp.float32),
        pltpu.SemaphoreType.DMA,
    ],
    compiler_params=pltpu.CompilerParams(use_tc_tiling_on_sc=False),
)
def _gather_sum(idx_hbm, tablet_hbm, out_hbm, idx_v, vals_v, acc_v, sem):
    wid = lax.axis_index("s") * NC + lax.axis_index("c")
    pltpu.sync_copy(idx_hbm, idx_v)
    row = tablet_hbm.at[wid]

    copies = [
        pltpu.async_copy(
            row.at[idx_v.at[j]],
            vals_v.at[pl.ds(j * ICH, ICH)],
            sem,
        )
        for j in range(NCHUNK)
    ]
    for c in copies:
        c.wait()

    def rbody(g, carry):
        o = g * 16
        acc = vals_v[pl.ds(o, 16)]
        for c in range(1, CTX):
            acc = acc + vals_v[pl.ds(c * BATCH + o, 16)]
        acc_v[pl.ds(o, 16)] = acc
        return carry

    lax.fori_loop(0, BATCH // 16, rbody, 0)
    pltpu.sync_copy(acc_v, out_hbm.at[wid])


VT = 2048
GRID = (VOCAB + VT - 1) // VT
NSPLIT = 4
CH = VT // NSPLIT
TAIL = VOCAB - (GRID - 1) * VT
TFULL = TAIL // CH
TREM = TAIL - TFULL * CH


def _proj_body(wt_ref, es_ref, b_ref, out_hbm, obuf, sem_o):
    i = pl.program_id(0)
    n = pl.num_programs(0)
    slot = lax.rem(i, 2)

    def chunk_copy(step, k, rows):
        s = lax.rem(step, 2)
        base = pl.multiple_of(step * VT, VT) + k * CH
        return pltpu.make_async_copy(
            obuf.at[s, pl.ds(k * CH, rows)],
            out_hbm.at[pl.ds(base, rows)],
            sem_o.at[s, k],
        )

    def start_out(step, last):
        nfull = TFULL if last else NSPLIT
        for k in range(nfull):
            chunk_copy(step, k, CH).start()
        if last:
            chunk_copy(step, TFULL, TREM).start()

    def drain_out(step, last):
        nfull = TFULL if last else NSPLIT
        for k in range(nfull):
            chunk_copy(step, k, CH).wait()
        if last:
            chunk_copy(step, TFULL, TREM).wait()

    @pl.when(i >= 2)
    def _():
        drain_out(i - 2, False)

    obuf[slot] = (
        lax.dot_general(
            wt_ref[...],
            es_ref[...],
            (((0,), (0,)), ((), ())),
            preferred_element_type=jnp.float32,
        )
        + b_ref[...][:, None]
    )

    @pl.when(i < n - 1)
    def _():
        start_out(i, False)

    @pl.when(i == n - 1)
    def _():
        start_out(i, True)
        drain_out(i - 1, False)
        drain_out(i, True)


def kernel(context_words, emb_table, W, b):
    idx = jnp.asarray(context_words, jnp.int32).reshape(NCHUNK, ICH)
    es_t = _gather_sum(idx, emb_table.T)

    out_t = pl.pallas_call(
        _proj_body,
        grid=(GRID,),
        in_specs=[
            pl.BlockSpec((D, VT), lambda i: (0, i)),
            pl.BlockSpec((D, BATCH), lambda i: (0, 0)),
            pl.BlockSpec((VT,), lambda i: (i,)),
        ],
        out_specs=pl.BlockSpec(memory_space=pl.ANY),
        out_shape=jax.ShapeDtypeStruct((VOCAB, BATCH), jnp.float32),
        scratch_shapes=[
            pltpu.VMEM((2, VT, BATCH), jnp.float32),
            pltpu.SemaphoreType.DMA((2, NSPLIT)),
        ],
    )(W.T, es_t, b)
    return out_t.T
